# baseline (speedup 1.0000x reference)
import jax
import jax.numpy as jnp
from jax import lax
from jax.experimental import pallas as pl
from jax.experimental.pallas import tpu as pltpu
from jax.experimental.pallas import tpu_sc as plsc

NN = 10000
EE = 320000
DD = 128
NC = 2
NS = 16
NW = NC * NS
CHUNK = 128
GRP = 8
NCH0 = 112
NCH1 = 48
NCHMAX = max(NCH0, NCH1)
N_PAD = 10240
ROWS_PT = N_PAD // NS
BLK = 2000

_mesh = plsc.VectorSubcoreMesh(core_axis_name="c", subcore_axis_name="s")


def _agg_body(h_hbm, src_hbm, dst_hbm, agg_out,
              src_v, dst_v, rows_a, rows_b, acc_sh, sem_a, sem_b):
    cid = lax.axis_index("c")
    sid = lax.axis_index("s")
    wid = cid * NS + sid
    base = sid * ROWS_PT
    ngrp = jnp.where(cid == 0, NCH0 // GRP, NCH1 // GRP)

    def zrow(r, _):
        for k in range(DD // 16):
            rows_a[r, pl.ds(k * 16, 16)] = jnp.zeros((16,), jnp.float32)
        return 0
    lax.fori_loop(0, CHUNK, zrow, 0)
    for b in range(ROWS_PT // CHUNK):
        pltpu.sync_copy(rows_a, acc_sh.at[pl.ds(base + b * CHUNK, CHUNK)])
    plsc.subcore_barrier()

    rows = (rows_a, rows_b)
    sems = (sem_a, sem_b)

    def group(g, _):
        gs = pl.ds(g * GRP, GRP)
        pltpu.sync_copy(src_hbm.at[wid, gs], src_v)
        pltpu.sync_copy(dst_hbm.at[wid, gs], dst_v)
        cps = [None, None]
        cps[0] = pltpu.async_copy(h_hbm.at[src_v.at[0]], rows[0], sems[0])
        for b in range(GRP):
            cur, nxt = b % 2, (b + 1) % 2
            cps[cur].wait()
            if b + 1 < GRP:
                cps[nxt] = pltpu.async_copy(
                    h_hbm.at[src_v.at[b + 1]], rows[nxt], sems[nxt])
            pltpu.sync_copy(rows[cur], acc_sh.at[dst_v.at[b]], add=True)
        return 0
    lax.fori_loop(0, ngrp, group, 0)
    plsc.subcore_barrier()

    pltpu.sync_copy(acc_sh.at[pl.ds(base, ROWS_PT)],
                    agg_out.at[cid, pl.ds(base, ROWS_PT)])


_agg = pl.kernel(
    _agg_body,
    out_type=jax.ShapeDtypeStruct((NC, N_PAD, DD), jnp.float32),
    mesh=_mesh,
    scratch_types=[
        pltpu.VMEM((GRP, CHUNK), jnp.int32),
        pltpu.VMEM((GRP, CHUNK), jnp.int32),
        pltpu.VMEM((CHUNK, DD), jnp.float32),
        pltpu.VMEM((CHUNK, DD), jnp.float32),
        pltpu.VMEM_SHARED((N_PAD, DD), jnp.float32),
        pltpu.SemaphoreType.DMA,
        pltpu.SemaphoreType.DMA,
    ],
)


def _hist_body(dst_hbm, hist_out, dst_v, ones_v, deg_sh):
    cid = lax.axis_index("c")
    sid = lax.axis_index("s")
    wid = cid * NS + sid
    base = sid * ROWS_PT
    ngrp = jnp.where(cid == 0, NCH0 // GRP, NCH1 // GRP)

    def zrow(r, _):
        for k in range(DD // 16):
            ones_v[r, pl.ds(k * 16, 16)] = jnp.zeros((16,), jnp.float32)
        return 0
    lax.fori_loop(0, CHUNK, zrow, 0)
    for b in range(ROWS_PT // CHUNK):
        pltpu.sync_copy(ones_v, deg_sh.at[pl.ds(base + b * CHUNK, CHUNK)])

    def orow(r, _):
        for k in range(DD // 16):
            ones_v[r, pl.ds(k * 16, 16)] = jnp.ones((16,), jnp.float32)
        return 0
    lax.fori_loop(0, CHUNK, orow, 0)
    plsc.subcore_barrier()

    def group(g, _):
        pltpu.sync_copy(dst_hbm.at[wid, pl.ds(g * GRP, GRP)], dst_v)
        for b in range(GRP):
            pltpu.sync_copy(ones_v, deg_sh.at[dst_v.at[b]], add=True)
        return 0
    lax.fori_loop(0, ngrp, group, 0)
    plsc.subcore_barrier()

    pltpu.sync_copy(deg_sh.at[pl.ds(base, ROWS_PT)],
                    hist_out.at[cid, pl.ds(base, ROWS_PT)])


_hist = pl.kernel(
    _hist_body,
    out_type=jax.ShapeDtypeStruct((NC, N_PAD, DD), jnp.float32),
    mesh=_mesh,
    scratch_types=[
        pltpu.VMEM((GRP, CHUNK), jnp.int32),
        pltpu.VMEM((CHUNK, DD), jnp.float32),
        pltpu.VMEM_SHARED((N_PAD, DD), jnp.float32),
    ],
)


def _invdeg_body(hist_a, hist_b, out):
    d = hist_a[0, :, 0:1] + hist_b[0, :, 0:1]
    out[...] = 1.0 / jnp.maximum(d, 1.0)


_invdeg = pl.pallas_call(
    _invdeg_body,
    grid=(1,),
    in_specs=[pl.BlockSpec((1, N_PAD, DD), lambda i: (0, 0, 0)),
              pl.BlockSpec((1, N_PAD, DD), lambda i: (1, 0, 0))],
    out_specs=pl.BlockSpec((N_PAD, 1), lambda i: (0, 0)),
    out_shape=jax.ShapeDtypeStruct((N_PAD, 1), jnp.float32),
)


def _dense_body(agg_a, agg_b, inv, h, wlt, bl, wrt, out):
    mean = (agg_a[0] + agg_b[0]) * inv[...]
    acc = jnp.dot(mean, wlt[...], preferred_element_type=jnp.float32)
    acc += bl[...]
    acc += jnp.dot(h[...], wrt[...], preferred_element_type=jnp.float32)
    out[...] = jnp.maximum(acc, 0.0)


def _final_body(agg_a, agg_b, inv, h, wlt, bl, wrt, lw, lb, out):
    mean = (agg_a[0] + agg_b[0]) * inv[...]
    acc = jnp.dot(mean, wlt[...], preferred_element_type=jnp.float32)
    acc += bl[...]
    acc += jnp.dot(h[...], wrt[...], preferred_element_type=jnp.float32)
    hn = jnp.maximum(acc, 0.0)
    out[...] = jnp.sum(hn * lw[...], axis=1, keepdims=True) + lb[0, 0]


_agg_a_spec = pl.BlockSpec((1, BLK, DD), lambda i: (0, i, 0))
_agg_b_spec = pl.BlockSpec((1, BLK, DD), lambda i: (1, i, 0))
_inv_spec = pl.BlockSpec((BLK, 1), lambda i: (i, 0))
_row_spec = pl.BlockSpec((BLK, DD), lambda i: (i, 0))
_w_spec = pl.BlockSpec((DD, DD), lambda i: (0, 0))
_b_spec = pl.BlockSpec((1, DD), lambda i: (0, 0))

_dense = pl.pallas_call(
    _dense_body,
    grid=(NN // BLK,),
    in_specs=[_agg_a_spec, _agg_b_spec, _inv_spec, _row_spec,
              _w_spec, _b_spec, _w_spec],
    out_specs=_row_spec,
    out_shape=jax.ShapeDtypeStruct((NN, DD), jnp.float32),
)

_final = pl.pallas_call(
    _final_body,
    grid=(NN // BLK,),
    in_specs=[_agg_a_spec, _agg_b_spec, _inv_spec, _row_spec,
              _w_spec, _b_spec, _w_spec, _b_spec,
              pl.BlockSpec((1, 1), lambda i: (0, 0))],
    out_specs=pl.BlockSpec((BLK, 1), lambda i: (i, 0)),
    out_shape=jax.ShapeDtypeStruct((NN, 1), jnp.float32),
)


def _split_edges(idx, fill):
    e0 = NS * NCH0 * CHUNK
    e1 = NS * NCH1 * CHUNK
    padded = jnp.concatenate(
        [idx, jnp.full((e0 + e1 - EE,), fill, jnp.int32)])
    c0 = padded[:e0].reshape(NS, NCH0, CHUNK)
    c1 = padded[e0:].reshape(NS, NCH1, CHUNK)
    c0 = jnp.pad(c0, ((0, 0), (0, NCHMAX - NCH0), (0, 0)),
                 constant_values=fill)
    c1 = jnp.pad(c1, ((0, 0), (0, NCHMAX - NCH1), (0, 0)),
                 constant_values=fill)
    return jnp.concatenate([c0, c1])


def kernel(x, edge_index, Wl0, bl0, Wr0, Wl1, bl1, Wr1, Wl2, bl2, Wr2,
           lin_W, lin_b):
    src_p = _split_edges(edge_index[0], 0)
    dst_p = _split_edges(edge_index[1], NN)

    hist = _hist(dst_p)
    inv = _invdeg(hist, hist)
    agg = _agg(x, src_p, dst_p)
    h = _dense(agg, agg, inv, x,
               Wl0.T, bl0.reshape(1, DD), Wr0.T)
    agg = _agg(h, src_p, dst_p)
    h = _dense(agg, agg, inv, h,
               Wl1.T, bl1.reshape(1, DD), Wr1.T)
    agg = _agg(h, src_p, dst_p)
    out = _final(agg, agg, inv, h,
                 Wl2.T, bl2.reshape(1, DD), Wr2.T,
                 lin_W.reshape(1, DD), lin_b.reshape(1, 1))
    return out.reshape(NN)


# speedup vs baseline: 1.0929x; 1.0929x over previous
import jax
import jax.numpy as jnp
from jax import lax
from jax.experimental import pallas as pl
from jax.experimental.pallas import tpu as pltpu
from jax.experimental.pallas import tpu_sc as plsc

NN = 10000
EE = 320000
DD = 128
NC = 2
NS = 16
NW = NC * NS
CHUNK = 128
GRP = 8
NCH0 = 120
NCH1 = 40
NCHMAX = max(NCH0, NCH1)
N_PAD = 10240
ROWS_PT = N_PAD // NS
BLK = 2000

_mesh = plsc.VectorSubcoreMesh(core_axis_name="c", subcore_axis_name="s")


def _agg_body(h_hbm, src_hbm, dst_hbm, agg_out,
              src_v, dst_v, rows_a, rows_b, acc_sh, sem_a, sem_b):
    cid = lax.axis_index("c")
    sid = lax.axis_index("s")
    wid = cid * NS + sid
    base = sid * ROWS_PT
    ngrp = jnp.where(cid == 0, NCH0 // GRP, NCH1 // GRP)

    def zrow(r, _):
        for k in range(DD // 16):
            rows_a[r, pl.ds(k * 16, 16)] = jnp.zeros((16,), jnp.float32)
        return 0
    lax.fori_loop(0, CHUNK, zrow, 0)
    for b in range(ROWS_PT // CHUNK):
        pltpu.sync_copy(rows_a, acc_sh.at[pl.ds(base + b * CHUNK, CHUNK)])
    plsc.subcore_barrier()

    rows = (rows_a, rows_b)
    sems = (sem_a, sem_b)

    def group(g, _):
        gs = pl.ds(g * GRP, GRP)
        pltpu.sync_copy(src_hbm.at[wid, gs], src_v)
        pltpu.sync_copy(dst_hbm.at[wid, gs], dst_v)
        cps = [None, None]
        cps[0] = pltpu.async_copy(h_hbm.at[src_v.at[0]], rows[0], sems[0])
        for b in range(GRP):
            cur, nxt = b % 2, (b + 1) % 2
            cps[cur].wait()
            if b + 1 < GRP:
                cps[nxt] = pltpu.async_copy(
                    h_hbm.at[src_v.at[b + 1]], rows[nxt], sems[nxt])
            pltpu.sync_copy(rows[cur], acc_sh.at[dst_v.at[b]], add=True)
        return 0
    lax.fori_loop(0, ngrp, group, 0)
    plsc.subcore_barrier()

    pltpu.sync_copy(acc_sh.at[pl.ds(base, ROWS_PT)],
                    agg_out.at[cid, pl.ds(base, ROWS_PT)])


_agg = pl.kernel(
    _agg_body,
    out_type=jax.ShapeDtypeStruct((NC, N_PAD, DD), jnp.float32),
    mesh=_mesh,
    scratch_types=[
        pltpu.VMEM((GRP, CHUNK), jnp.int32),
        pltpu.VMEM((GRP, CHUNK), jnp.int32),
        pltpu.VMEM((CHUNK, DD), jnp.float32),
        pltpu.VMEM((CHUNK, DD), jnp.float32),
        pltpu.VMEM_SHARED((N_PAD, DD), jnp.float32),
        pltpu.SemaphoreType.DMA,
        pltpu.SemaphoreType.DMA,
    ],
)


def _hist_body(dst_hbm, hist_out, dst_v, ones_v, deg_sh):
    cid = lax.axis_index("c")
    sid = lax.axis_index("s")
    wid = cid * NS + sid
    base = sid * ROWS_PT
    ngrp = jnp.where(cid == 0, NCH0 // GRP, NCH1 // GRP)

    def zrow(r, _):
        for k in range(DD // 16):
            ones_v[r, pl.ds(k * 16, 16)] = jnp.zeros((16,), jnp.float32)
        return 0
    lax.fori_loop(0, CHUNK, zrow, 0)
    for b in range(ROWS_PT // CHUNK):
        pltpu.sync_copy(ones_v, deg_sh.at[pl.ds(base + b * CHUNK, CHUNK)])

    def orow(r, _):
        for k in range(DD // 16):
            ones_v[r, pl.ds(k * 16, 16)] = jnp.ones((16,), jnp.float32)
        return 0
    lax.fori_loop(0, CHUNK, orow, 0)
    plsc.subcore_barrier()

    def group(g, _):
        pltpu.sync_copy(dst_hbm.at[wid, pl.ds(g * GRP, GRP)], dst_v)
        for b in range(GRP):
            pltpu.sync_copy(ones_v, deg_sh.at[dst_v.at[b]], add=True)
        return 0
    lax.fori_loop(0, ngrp, group, 0)
    plsc.subcore_barrier()

    pltpu.sync_copy(deg_sh.at[pl.ds(base, ROWS_PT)],
                    hist_out.at[cid, pl.ds(base, ROWS_PT)])


_hist = pl.kernel(
    _hist_body,
    out_type=jax.ShapeDtypeStruct((NC, N_PAD, DD), jnp.float32),
    mesh=_mesh,
    scratch_types=[
        pltpu.VMEM((GRP, CHUNK), jnp.int32),
        pltpu.VMEM((CHUNK, DD), jnp.float32),
        pltpu.VMEM_SHARED((N_PAD, DD), jnp.float32),
    ],
)


def _invdeg_body(hist_a, hist_b, out):
    d = hist_a[0, :, 0:1] + hist_b[0, :, 0:1]
    out[...] = 1.0 / jnp.maximum(d, 1.0)


_invdeg = pl.pallas_call(
    _invdeg_body,
    grid=(1,),
    in_specs=[pl.BlockSpec((1, N_PAD, DD), lambda i: (0, 0, 0)),
              pl.BlockSpec((1, N_PAD, DD), lambda i: (1, 0, 0))],
    out_specs=pl.BlockSpec((N_PAD, 1), lambda i: (0, 0)),
    out_shape=jax.ShapeDtypeStruct((N_PAD, 1), jnp.float32),
)


def _dense_body(agg_a, agg_b, inv, h, wlt, bl, wrt, out):
    mean = (agg_a[0] + agg_b[0]) * inv[...]
    acc = jnp.dot(mean, wlt[...], preferred_element_type=jnp.float32)
    acc += bl[...]
    acc += jnp.dot(h[...], wrt[...], preferred_element_type=jnp.float32)
    out[...] = jnp.maximum(acc, 0.0)


def _final_body(agg_a, agg_b, inv, h, wlt, bl, wrt, lw, lb, out):
    mean = (agg_a[0] + agg_b[0]) * inv[...]
    acc = jnp.dot(mean, wlt[...], preferred_element_type=jnp.float32)
    acc += bl[...]
    acc += jnp.dot(h[...], wrt[...], preferred_element_type=jnp.float32)
    hn = jnp.maximum(acc, 0.0)
    out[...] = jnp.sum(hn * lw[...], axis=1, keepdims=True) + lb[0, 0]


_agg_a_spec = pl.BlockSpec((1, BLK, DD), lambda i: (0, i, 0))
_agg_b_spec = pl.BlockSpec((1, BLK, DD), lambda i: (1, i, 0))
_inv_spec = pl.BlockSpec((BLK, 1), lambda i: (i, 0))
_row_spec = pl.BlockSpec((BLK, DD), lambda i: (i, 0))
_w_spec = pl.BlockSpec((DD, DD), lambda i: (0, 0))
_b_spec = pl.BlockSpec((1, DD), lambda i: (0, 0))

_dense = pl.pallas_call(
    _dense_body,
    grid=(NN // BLK,),
    in_specs=[_agg_a_spec, _agg_b_spec, _inv_spec, _row_spec,
              _w_spec, _b_spec, _w_spec],
    out_specs=_row_spec,
    out_shape=jax.ShapeDtypeStruct((NN, DD), jnp.float32),
)

_final = pl.pallas_call(
    _final_body,
    grid=(NN // BLK,),
    in_specs=[_agg_a_spec, _agg_b_spec, _inv_spec, _row_spec,
              _w_spec, _b_spec, _w_spec, _b_spec,
              pl.BlockSpec((1, 1), lambda i: (0, 0))],
    out_specs=pl.BlockSpec((BLK, 1), lambda i: (i, 0)),
    out_shape=jax.ShapeDtypeStruct((NN, 1), jnp.float32),
)


def _split_edges(idx, fill):
    e0 = NS * NCH0 * CHUNK
    e1 = NS * NCH1 * CHUNK
    padded = jnp.concatenate(
        [idx, jnp.full((e0 + e1 - EE,), fill, jnp.int32)])
    c0 = padded[:e0].reshape(NS, NCH0, CHUNK)
    c1 = padded[e0:].reshape(NS, NCH1, CHUNK)
    c0 = jnp.pad(c0, ((0, 0), (0, NCHMAX - NCH0), (0, 0)),
                 constant_values=fill)
    c1 = jnp.pad(c1, ((0, 0), (0, NCHMAX - NCH1), (0, 0)),
                 constant_values=fill)
    return jnp.concatenate([c0, c1])


def kernel(x, edge_index, Wl0, bl0, Wr0, Wl1, bl1, Wr1, Wl2, bl2, Wr2,
           lin_W, lin_b):
    src_p = _split_edges(edge_index[0], 0)
    dst_p = _split_edges(edge_index[1], NN)

    hist = _hist(dst_p)
    inv = _invdeg(hist, hist)
    agg = _agg(x, src_p, dst_p)
    h = _dense(agg, agg, inv, x,
               Wl0.T, bl0.reshape(1, DD), Wr0.T)
    agg = _agg(h, src_p, dst_p)
    h = _dense(agg, agg, inv, h,
               Wl1.T, bl1.reshape(1, DD), Wr1.T)
    agg = _agg(h, src_p, dst_p)
    out = _final(agg, agg, inv, h,
                 Wl2.T, bl2.reshape(1, DD), Wr2.T,
                 lin_W.reshape(1, DD), lin_b.reshape(1, 1))
    return out.reshape(NN)
